# baseline (device time: 46335 ns/iter reference)
import jax
import jax.numpy as jnp
from jax import lax
from jax.experimental import pallas as pl
from jax.experimental.pallas import tpu as pltpu

N_DEV = 4


def kernel(x):
    x = x.reshape(x.shape[-2], x.shape[-1])
    m, n = x.shape
    H, Q, E = m // 2, m // 4, m // 8

    def body(x_ref, out_ref, cA1, cB1, cA2, cB2, send_sems, recv_sems):
        p = lax.axis_index("i")
        my_x = p // 2
        my_y = (p % 2) ^ my_x
        ox, oy = 1 - my_x, 1 - my_y
        xp = 3 - p
        yp = p ^ 1

        barrier_sem = pltpu.get_barrier_semaphore()
        for nbr in [xp, yp]:
            pl.semaphore_signal(
                barrier_sem, inc=1,
                device_id=(nbr,), device_id_type=pl.DeviceIdType.MESH,
            )
        pl.semaphore_wait(barrier_sem, 2)

        def rc(src, dst, sem_i, dev):
            return pltpu.make_async_remote_copy(
                src_ref=src, dst_ref=dst,
                send_sem=send_sems.at[sem_i], recv_sem=recv_sems.at[sem_i],
                device_id=(dev,), device_id_type=pl.DeviceIdType.MESH,
            )

        a_half = my_y * Q
        a_keep = a_half + my_x * E
        a_send = a_half + ox * E
        b_half = H + my_x * Q
        b_keep = b_half + my_y * E
        b_send = b_half + oy * E

        p1a = rc(x_ref.at[pl.ds(oy * Q, Q)], cA1, 0, yp)
        p1b = rc(x_ref.at[pl.ds(H + ox * Q, Q)], cB1, 1, xp)
        p1a.start()
        p1b.start()
        p1a.wait()
        p1b.wait()

        out_ref[pl.ds(a_send, E), :] = (
            x_ref[pl.ds(a_send, E), :] + cA1[pl.ds(ox * E, E), :]
        )
        out_ref[pl.ds(b_send, E), :] = (
            x_ref[pl.ds(b_send, E), :] + cB1[pl.ds(oy * E, E), :]
        )

        p2a = rc(out_ref.at[pl.ds(a_send, E)], cA2, 2, xp)
        p2b = rc(out_ref.at[pl.ds(b_send, E)], cB2, 3, yp)
        p2a.start()
        p2b.start()
        out_ref[pl.ds(a_keep, E), :] = (
            x_ref[pl.ds(a_keep, E), :] + cA1[pl.ds(my_x * E, E), :]
        )
        out_ref[pl.ds(b_keep, E), :] = (
            x_ref[pl.ds(b_keep, E), :] + cB1[pl.ds(my_y * E, E), :]
        )
        p2a.wait()
        p2b.wait()
        out_ref[pl.ds(a_keep, E), :] += cA2[...]
        out_ref[pl.ds(b_keep, E), :] += cB2[...]

        p3a = rc(out_ref.at[pl.ds(a_keep, E)], out_ref.at[pl.ds(a_keep, E)],
                 4, xp)
        p3b = rc(out_ref.at[pl.ds(b_keep, E)], out_ref.at[pl.ds(b_keep, E)],
                 5, yp)
        p4aa = rc(out_ref.at[pl.ds(a_keep, E)], out_ref.at[pl.ds(a_keep, E)],
                  6, yp)
        p4ab = rc(out_ref.at[pl.ds(b_keep, E)], out_ref.at[pl.ds(b_keep, E)],
                  8, xp)
        p3a.start()
        p3b.start()
        p4aa.start()
        p4ab.start()
        p3a.wait()
        p3b.wait()

        p4ba = rc(out_ref.at[pl.ds(a_send, E)], out_ref.at[pl.ds(a_send, E)],
                  7, yp)
        p4bb = rc(out_ref.at[pl.ds(b_send, E)], out_ref.at[pl.ds(b_send, E)],
                  9, xp)
        p4ba.start()
        p4bb.start()
        p4aa.wait()
        p4ab.wait()
        p4ba.wait()
        p4bb.wait()

    return pl.pallas_call(
        body,
        out_shape=jax.ShapeDtypeStruct((m, n), jnp.float32),
        in_specs=[pl.BlockSpec(memory_space=pltpu.VMEM)],
        out_specs=pl.BlockSpec(memory_space=pltpu.VMEM),
        scratch_shapes=[
            pltpu.VMEM((Q, n), jnp.float32),
            pltpu.VMEM((Q, n), jnp.float32),
            pltpu.VMEM((E, n), jnp.float32),
            pltpu.VMEM((E, n), jnp.float32),
            pltpu.SemaphoreType.DMA((10,)),
            pltpu.SemaphoreType.DMA((10,)),
        ],
        compiler_params=pltpu.CompilerParams(collective_id=0),
    )(x)


# device time: 45235 ns/iter; 1.0243x vs baseline; 1.0243x over previous
import jax
import jax.numpy as jnp
from jax import lax
from jax.experimental import pallas as pl
from jax.experimental.pallas import tpu as pltpu

N_DEV = 4


def kernel(x):
    x = x.reshape(x.shape[-2], x.shape[-1])
    m, n = x.shape
    H, Q, E = m // 2, m // 4, m // 8

    def body(x_ref, out_ref, xa, xb, cA1, cB1, cA2, cB2,
             send_sems, recv_sems, local_sems):
        p = lax.axis_index("i")
        my_x = p // 2
        my_y = (p % 2) ^ my_x
        ox, oy = 1 - my_x, 1 - my_y
        xp = 3 - p
        yp = p ^ 1

        barrier_sem = pltpu.get_barrier_semaphore()
        for nbr in [xp, yp]:
            pl.semaphore_signal(
                barrier_sem, inc=1,
                device_id=(nbr,), device_id_type=pl.DeviceIdType.MESH,
            )
        pl.semaphore_wait(barrier_sem, 2)

        def rc(src, dst, sem_i, dev):
            return pltpu.make_async_remote_copy(
                src_ref=src, dst_ref=dst,
                send_sem=send_sems.at[sem_i], recv_sem=recv_sems.at[sem_i],
                device_id=(dev,), device_id_type=pl.DeviceIdType.MESH,
            )

        a_keep = my_y * Q + my_x * E
        a_send = my_y * Q + ox * E
        b_keep = H + my_x * Q + my_y * E
        b_send = H + my_x * Q + oy * E

        p1a = rc(x_ref.at[pl.ds(oy * Q, Q)], cA1, 0, yp)
        p1b = rc(x_ref.at[pl.ds(H + ox * Q, Q)], cB1, 1, xp)
        p1a.start()
        p1b.start()
        lda = pltpu.make_async_copy(
            x_ref.at[pl.ds(my_y * Q, Q)], xa, local_sems.at[0])
        ldb = pltpu.make_async_copy(
            x_ref.at[pl.ds(H + my_x * Q, Q)], xb, local_sems.at[1])
        lda.start()
        ldb.start()
        lda.wait()
        ldb.wait()
        p1a.wait()
        p1b.wait()

        xa[pl.ds(ox * E, E), :] += cA1[pl.ds(ox * E, E), :]
        xb[pl.ds(oy * E, E), :] += cB1[pl.ds(oy * E, E), :]

        p2a = rc(xa.at[pl.ds(ox * E, E)], cA2, 2, xp)
        p2b = rc(xb.at[pl.ds(oy * E, E)], cB2, 3, yp)
        p2a.start()
        p2b.start()
        xa[pl.ds(my_x * E, E), :] += cA1[pl.ds(my_x * E, E), :]
        xb[pl.ds(my_y * E, E), :] += cB1[pl.ds(my_y * E, E), :]
        p2a.wait()
        p2b.wait()
        xa[pl.ds(my_x * E, E), :] += cA2[...]
        xb[pl.ds(my_y * E, E), :] += cB2[...]

        p3a = rc(xa.at[pl.ds(my_x * E, E)], out_ref.at[pl.ds(a_keep, E)],
                 4, xp)
        p3b = rc(xb.at[pl.ds(my_y * E, E)], out_ref.at[pl.ds(b_keep, E)],
                 5, yp)
        p4aa = rc(xa.at[pl.ds(my_x * E, E)], out_ref.at[pl.ds(a_keep, E)],
                  6, yp)
        p4ab = rc(xb.at[pl.ds(my_y * E, E)], out_ref.at[pl.ds(b_keep, E)],
                  8, xp)
        p3a.start()
        p3b.start()
        p4aa.start()
        p4ab.start()
        sta = pltpu.make_async_copy(
            xa.at[pl.ds(my_x * E, E)], out_ref.at[pl.ds(a_keep, E)],
            local_sems.at[2])
        stb = pltpu.make_async_copy(
            xb.at[pl.ds(my_y * E, E)], out_ref.at[pl.ds(b_keep, E)],
            local_sems.at[3])
        sta.start()
        stb.start()
        p3a.wait()
        p3b.wait()

        p4ba = rc(out_ref.at[pl.ds(a_send, E)], out_ref.at[pl.ds(a_send, E)],
                  7, yp)
        p4bb = rc(out_ref.at[pl.ds(b_send, E)], out_ref.at[pl.ds(b_send, E)],
                  9, xp)
        p4ba.start()
        p4bb.start()
        sta.wait()
        stb.wait()
        p4aa.wait()
        p4ab.wait()
        p4ba.wait()
        p4bb.wait()

    return pl.pallas_call(
        body,
        out_shape=jax.ShapeDtypeStruct((m, n), jnp.float32),
        in_specs=[pl.BlockSpec(memory_space=pltpu.MemorySpace.HBM)],
        out_specs=pl.BlockSpec(memory_space=pltpu.MemorySpace.HBM),
        scratch_shapes=[
            pltpu.VMEM((Q, n), jnp.float32),
            pltpu.VMEM((Q, n), jnp.float32),
            pltpu.VMEM((Q, n), jnp.float32),
            pltpu.VMEM((Q, n), jnp.float32),
            pltpu.VMEM((E, n), jnp.float32),
            pltpu.VMEM((E, n), jnp.float32),
            pltpu.SemaphoreType.DMA((10,)),
            pltpu.SemaphoreType.DMA((10,)),
            pltpu.SemaphoreType.DMA((4,)),
        ],
        compiler_params=pltpu.CompilerParams(collective_id=0),
    )(x)


# device time: 44216 ns/iter; 1.0479x vs baseline; 1.0230x over previous
import jax
import jax.numpy as jnp
from jax import lax
from jax.experimental import pallas as pl
from jax.experimental.pallas import tpu as pltpu

N_DEV = 4


def kernel(x):
    _, m, n = x.shape
    H, Q, E = m // 2, m // 4, m // 8

    def body(x_ref, out_ref, xa, xb, cA1, cB1, cA2, cB2,
             send_sems, recv_sems, local_sems):
        p = lax.axis_index("i")
        my_x = p // 2
        my_y = (p % 2) ^ my_x
        ox, oy = 1 - my_x, 1 - my_y
        xp = 3 - p
        yp = p ^ 1

        barrier_sem = pltpu.get_barrier_semaphore()
        for nbr in [xp, yp]:
            pl.semaphore_signal(
                barrier_sem, inc=1,
                device_id=(nbr,), device_id_type=pl.DeviceIdType.MESH,
            )
        pl.semaphore_wait(barrier_sem, 2)

        def rc(src, dst, sem_i, dev):
            return pltpu.make_async_remote_copy(
                src_ref=src, dst_ref=dst,
                send_sem=send_sems.at[sem_i], recv_sem=recv_sems.at[sem_i],
                device_id=(dev,), device_id_type=pl.DeviceIdType.MESH,
            )

        a_keep = my_y * Q + my_x * E
        a_send = my_y * Q + ox * E
        b_keep = H + my_x * Q + my_y * E
        b_send = H + my_x * Q + oy * E

        p1a1 = rc(x_ref.at[0, pl.ds(oy * Q + ox * E, E)],
                  cA1.at[pl.ds(ox * E, E)], 0, yp)
        p1b1 = rc(x_ref.at[0, pl.ds(H + ox * Q + oy * E, E)],
                  cB1.at[pl.ds(oy * E, E)], 1, xp)
        p1a2 = rc(x_ref.at[0, pl.ds(oy * Q + my_x * E, E)],
                  cA1.at[pl.ds(my_x * E, E)], 10, yp)
        p1b2 = rc(x_ref.at[0, pl.ds(H + ox * Q + my_y * E, E)],
                  cB1.at[pl.ds(my_y * E, E)], 11, xp)
        p1a1.start()
        p1b1.start()
        p1a2.start()
        p1b2.start()
        lda = pltpu.make_async_copy(
            x_ref.at[0, pl.ds(my_y * Q, Q)], xa, local_sems.at[0])
        ldb = pltpu.make_async_copy(
            x_ref.at[0, pl.ds(H + my_x * Q, Q)], xb, local_sems.at[1])
        lda.start()
        ldb.start()
        lda.wait()
        ldb.wait()
        p1a1.wait()
        p1b1.wait()

        xa[pl.ds(ox * E, E), :] += cA1[pl.ds(ox * E, E), :]
        xb[pl.ds(oy * E, E), :] += cB1[pl.ds(oy * E, E), :]
        p2a = rc(xa.at[pl.ds(ox * E, E)], cA2, 2, xp)
        p2b = rc(xb.at[pl.ds(oy * E, E)], cB2, 3, yp)
        p2a.start()
        p2b.start()
        p1a2.wait()
        p1b2.wait()
        xa[pl.ds(my_x * E, E), :] += cA1[pl.ds(my_x * E, E), :]
        xb[pl.ds(my_y * E, E), :] += cB1[pl.ds(my_y * E, E), :]
        p2a.wait()
        p2b.wait()
        xa[pl.ds(my_x * E, E), :] += cA2[...]
        xb[pl.ds(my_y * E, E), :] += cB2[...]

        p3a = rc(xa.at[pl.ds(my_x * E, E)], out_ref.at[pl.ds(a_keep, E)],
                 4, xp)
        p3b = rc(xb.at[pl.ds(my_y * E, E)], out_ref.at[pl.ds(b_keep, E)],
                 5, yp)
        p4aa = rc(xa.at[pl.ds(my_x * E, E)], out_ref.at[pl.ds(a_keep, E)],
                  6, yp)
        p4ab = rc(xb.at[pl.ds(my_y * E, E)], out_ref.at[pl.ds(b_keep, E)],
                  8, xp)
        p3a.start()
        p3b.start()
        p4aa.start()
        p4ab.start()
        sta = pltpu.make_async_copy(
            xa.at[pl.ds(my_x * E, E)], out_ref.at[pl.ds(a_keep, E)],
            local_sems.at[2])
        stb = pltpu.make_async_copy(
            xb.at[pl.ds(my_y * E, E)], out_ref.at[pl.ds(b_keep, E)],
            local_sems.at[3])
        sta.start()
        stb.start()
        p3a.wait()
        p3b.wait()

        p4ba = rc(out_ref.at[pl.ds(a_send, E)], out_ref.at[pl.ds(a_send, E)],
                  7, yp)
        p4bb = rc(out_ref.at[pl.ds(b_send, E)], out_ref.at[pl.ds(b_send, E)],
                  9, xp)
        p4ba.start()
        p4bb.start()
        sta.wait()
        stb.wait()
        p4aa.wait()
        p4ab.wait()
        p4ba.wait()
        p4bb.wait()

    return pl.pallas_call(
        body,
        out_shape=jax.ShapeDtypeStruct((m, n), jnp.float32),
        in_specs=[pl.BlockSpec(memory_space=pltpu.MemorySpace.HBM)],
        out_specs=pl.BlockSpec(memory_space=pltpu.MemorySpace.HBM),
        scratch_shapes=[
            pltpu.VMEM((Q, n), jnp.float32),
            pltpu.VMEM((Q, n), jnp.float32),
            pltpu.VMEM((Q, n), jnp.float32),
            pltpu.VMEM((Q, n), jnp.float32),
            pltpu.VMEM((E, n), jnp.float32),
            pltpu.VMEM((E, n), jnp.float32),
            pltpu.SemaphoreType.DMA((12,)),
            pltpu.SemaphoreType.DMA((12,)),
            pltpu.SemaphoreType.DMA((4,)),
        ],
        compiler_params=pltpu.CompilerParams(collective_id=0),
    )(x)


# device time: 43057 ns/iter; 1.0761x vs baseline; 1.0269x over previous
import jax
import jax.numpy as jnp
from jax import lax
from jax.experimental import pallas as pl
from jax.experimental.pallas import tpu as pltpu

N_DEV = 4


def kernel(x):
    _, m, n = x.shape
    H, Q, E = m // 2, m // 4, m // 8
    F = m // 16

    def body(x_ref, out_ref, xa, xb, cA1, cB1, cA2, cB2,
             send_sems, recv_sems, local_sems):
        p = lax.axis_index("i")
        my_x = p // 2
        my_y = (p % 2) ^ my_x
        ox, oy = 1 - my_x, 1 - my_y
        xp = 3 - p
        yp = p ^ 1

        lda = pltpu.make_async_copy(
            x_ref.at[0, pl.ds(my_y * Q, Q)], xa, local_sems.at[0])
        ldb = pltpu.make_async_copy(
            x_ref.at[0, pl.ds(H + my_x * Q, Q)], xb, local_sems.at[1])
        lda.start()
        ldb.start()

        barrier_sem = pltpu.get_barrier_semaphore()
        for nbr in [xp, yp]:
            pl.semaphore_signal(
                barrier_sem, inc=1,
                device_id=(nbr,), device_id_type=pl.DeviceIdType.MESH,
            )
        pl.semaphore_wait(barrier_sem, 2)

        def rc(src, dst, sem_i, dev):
            return pltpu.make_async_remote_copy(
                src_ref=src, dst_ref=dst,
                send_sem=send_sems.at[sem_i], recv_sem=recv_sems.at[sem_i],
                device_id=(dev,), device_id_type=pl.DeviceIdType.MESH,
            )

        a_keep = my_y * Q + my_x * E
        a_send = my_y * Q + ox * E
        b_keep = H + my_x * Q + my_y * E
        b_send = H + my_x * Q + oy * E

        p1a1 = rc(x_ref.at[0, pl.ds(oy * Q + ox * E, E)],
                  cA1.at[pl.ds(ox * E, E)], 0, yp)
        p1b1 = rc(x_ref.at[0, pl.ds(H + ox * Q + oy * E, E)],
                  cB1.at[pl.ds(oy * E, E)], 1, xp)
        p1a2 = rc(x_ref.at[0, pl.ds(oy * Q + my_x * E, E)],
                  cA1.at[pl.ds(my_x * E, E)], 10, yp)
        p1b2 = rc(x_ref.at[0, pl.ds(H + ox * Q + my_y * E, E)],
                  cB1.at[pl.ds(my_y * E, E)], 11, xp)
        p1a1.start()
        p1b1.start()
        p1a2.start()
        p1b2.start()
        lda.wait()
        ldb.wait()
        p1a1.wait()
        p1b1.wait()

        xa[pl.ds(ox * E, E), :] += cA1[pl.ds(ox * E, E), :]
        xb[pl.ds(oy * E, E), :] += cB1[pl.ds(oy * E, E), :]
        p2a1 = rc(xa.at[pl.ds(ox * E, F)], cA2.at[pl.ds(0, F)], 2, xp)
        p2a2 = rc(xa.at[pl.ds(ox * E + F, F)], cA2.at[pl.ds(F, F)], 12, xp)
        p2b1 = rc(xb.at[pl.ds(oy * E, F)], cB2.at[pl.ds(0, F)], 3, yp)
        p2b2 = rc(xb.at[pl.ds(oy * E + F, F)], cB2.at[pl.ds(F, F)], 13, yp)
        p2a1.start()
        p2a2.start()
        p2b1.start()
        p2b2.start()
        p1a2.wait()
        p1b2.wait()
        xa[pl.ds(my_x * E, E), :] += cA1[pl.ds(my_x * E, E), :]
        xb[pl.ds(my_y * E, E), :] += cB1[pl.ds(my_y * E, E), :]

        p2a1.wait()
        p2b1.wait()
        xa[pl.ds(my_x * E, F), :] += cA2[pl.ds(0, F), :]
        xb[pl.ds(my_y * E, F), :] += cB2[pl.ds(0, F), :]
        p3a1 = rc(xa.at[pl.ds(my_x * E, F)], out_ref.at[pl.ds(a_keep, F)],
                  4, xp)
        p3b1 = rc(xb.at[pl.ds(my_y * E, F)], out_ref.at[pl.ds(b_keep, F)],
                  5, yp)
        p3a1.start()
        p3b1.start()
        p2a2.wait()
        p2b2.wait()
        xa[pl.ds(my_x * E + F, F), :] += cA2[pl.ds(F, F), :]
        xb[pl.ds(my_y * E + F, F), :] += cB2[pl.ds(F, F), :]
        p3a2 = rc(xa.at[pl.ds(my_x * E + F, F)],
                  out_ref.at[pl.ds(a_keep + F, F)], 14, xp)
        p3b2 = rc(xb.at[pl.ds(my_y * E + F, F)],
                  out_ref.at[pl.ds(b_keep + F, F)], 15, yp)
        p4aa = rc(xa.at[pl.ds(my_x * E, E)], out_ref.at[pl.ds(a_keep, E)],
                  6, yp)
        p4ab = rc(xb.at[pl.ds(my_y * E, E)], out_ref.at[pl.ds(b_keep, E)],
                  8, xp)
        p3a2.start()
        p3b2.start()
        p4aa.start()
        p4ab.start()
        sta = pltpu.make_async_copy(
            xa.at[pl.ds(my_x * E, E)], out_ref.at[pl.ds(a_keep, E)],
            local_sems.at[2])
        stb = pltpu.make_async_copy(
            xb.at[pl.ds(my_y * E, E)], out_ref.at[pl.ds(b_keep, E)],
            local_sems.at[3])
        sta.start()
        stb.start()
        p3a1.wait()
        p3b1.wait()
        p3a2.wait()
        p3b2.wait()

        p4ba = rc(out_ref.at[pl.ds(a_send, E)], out_ref.at[pl.ds(a_send, E)],
                  7, yp)
        p4bb = rc(out_ref.at[pl.ds(b_send, E)], out_ref.at[pl.ds(b_send, E)],
                  9, xp)
        p4ba.start()
        p4bb.start()
        sta.wait()
        stb.wait()
        p4aa.wait()
        p4ab.wait()
        p4ba.wait()
        p4bb.wait()

    return pl.pallas_call(
        body,
        out_shape=jax.ShapeDtypeStruct((m, n), jnp.float32),
        in_specs=[pl.BlockSpec(memory_space=pltpu.MemorySpace.HBM)],
        out_specs=pl.BlockSpec(memory_space=pltpu.MemorySpace.HBM),
        scratch_shapes=[
            pltpu.VMEM((Q, n), jnp.float32),
            pltpu.VMEM((Q, n), jnp.float32),
            pltpu.VMEM((Q, n), jnp.float32),
            pltpu.VMEM((Q, n), jnp.float32),
            pltpu.VMEM((E, n), jnp.float32),
            pltpu.VMEM((E, n), jnp.float32),
            pltpu.SemaphoreType.DMA((16,)),
            pltpu.SemaphoreType.DMA((16,)),
            pltpu.SemaphoreType.DMA((4,)),
        ],
        compiler_params=pltpu.CompilerParams(collective_id=0),
    )(x)
